# baseline (speedup 1.0000x reference)
import functools

import jax
import jax.numpy as jnp
from jax import lax
from jax.experimental import pallas as pl
from jax.experimental.pallas import tpu as pltpu
from jax.experimental.pallas import tpu_sc as plsc

B, S, D, M, V = 64, 50, 128, 2048, 100000
TOPK = 12
NW = 32
ROWS_PER_W = B // NW
LANES = 16
KBW = 64


BBLK = 64


def _ivec_body(dh_ref, h_ref, w1_ref, b1_ref, w2_ref, b2_ref, out_ref):
    x = dh_ref[...]
    h = h_ref[0]
    hb = jnp.broadcast_to(h[:, None, :], (BBLK, S, D))
    cat = jnp.concatenate([hb, x], axis=2).reshape(BBLK * S, 2 * D)
    t = jnp.tanh(jnp.dot(cat, w1_ref[...],
                         preferred_element_type=jnp.float32) + b1_ref[...])
    q = (jnp.dot(t, w2_ref[...],
                 preferred_element_type=jnp.float32) + b2_ref[...])
    q = q.reshape(BBLK, S, D)
    q = q - jnp.max(q, axis=1, keepdims=True)
    e = jnp.exp(q)
    q = e / jnp.sum(e, axis=1, keepdims=True)
    out_ref[...] = jnp.sum(q * x, axis=1)


def _pick_own_batch(R, n):
    R3 = R.reshape(B, n, B)
    bb = lax.broadcasted_iota(jnp.int32, (B, n, B), 0)
    jb = lax.broadcasted_iota(jnp.int32, (B, n, B), 2)
    return jnp.sum(jnp.where(jb == bb, R3, 0.0), axis=2)


def _final_body(rows_ref, roww_ref, gp_ref, kb_ref, ctx_ref, iv_ref, out_ref):
    iv = iv_ref[...]
    gpp = gp_ref[...]
    rowsS = rows_ref[...] * gpp[:, :KBW, None]
    rows2 = rowsS.reshape(B * KBW, D)
    R = lax.dot_general(rows2, iv, (((1,), (1,)), ((), ())))
    out64 = _pick_own_batch(R, KBW)
    rowwS = roww_ref[...] * gpp[:, KBW:KBW + 1, None]
    roww2 = rowwS.reshape(B * LANES, D)
    Rw = lax.dot_general(roww2, iv, (((1,), (1,)), ((), ())))
    outw = _pick_own_batch(Rw, LANES)[:, :1]

    kb = kb_ref[:, :1]
    ctx = ctx_ref[:, :1]
    pos64 = lax.broadcasted_iota(jnp.int32, (B, KBW), 1)
    bad64 = ((pos64 >= kb) & (pos64 < ctx - 1)) | (pos64 >= ctx)
    x64 = jnp.where(bad64, jnp.float32(-1e9), out64)
    sig64 = 1.0 / (1.0 + jnp.exp(-x64))
    win = 1.0 / (1.0 + jnp.exp(-outw))

    l = jnp.concatenate(
        [sig64, win, jnp.zeros((B, KBW - 1), jnp.float32)], axis=1)
    rawpos = lax.broadcasted_iota(jnp.int32, (B, 2 * KBW), 1)
    pos = jnp.where(rawpos == KBW, ctx - 1, rawpos)
    cols = []
    for _ in range(TOPK):
        v = jnp.max(l, axis=1, keepdims=True)
        idx = jnp.min(jnp.where(l == v, pos, M), axis=1, keepdims=True)
        cols.append(idx)
        l = jnp.where(pos == idx, -jnp.inf, l)
    out_ref[...] = jnp.concatenate(cols, axis=1)


def _sc_gather(c_know, story, gp, ctx_len):
    mesh = plsc.VectorSubcoreMesh(core_axis_name="c", subcore_axis_name="s")

    @functools.partial(
        pl.kernel, mesh=mesh,
        out_type=(jax.ShapeDtypeStruct((B, KBW, D), jnp.float32),
                  jax.ShapeDtypeStruct((B, LANES, D), jnp.float32),
                  jax.ShapeDtypeStruct((B, 2 * KBW), jnp.float32)),
        scratch_types=(
            [pltpu.VMEM((KBW,), jnp.int32)] * ROWS_PER_W
            + [pltpu.VMEM((LANES,), jnp.int32)] * ROWS_PER_W
            + [pltpu.VMEM((LANES,), jnp.int32)] * ROWS_PER_W
            + [pltpu.VMEM((KBW,), jnp.float32)] * ROWS_PER_W
            + [pltpu.VMEM((LANES,), jnp.float32)] * ROWS_PER_W
            + [pltpu.VMEM((KBW, D), jnp.float32)] * ROWS_PER_W
            + [pltpu.VMEM((LANES, D), jnp.float32)] * ROWS_PER_W
            + [pltpu.VMEM((2 * KBW,), jnp.float32)] * ROWS_PER_W
            + [pltpu.VMEM((B, LANES), jnp.int32)]
            + [pltpu.SemaphoreType.DMA] * (2 * ROWS_PER_W)
            + [pltpu.SemaphoreType.DMA] * ROWS_PER_W
            + [pltpu.SemaphoreType.DMA] * ROWS_PER_W
            + [pltpu.SemaphoreType.DMA]
        ),
    )
    def k(c_hbm, story_hbm, gp_hbm, ctx_hbm, rows_hbm, roww_hbm, gpp_hbm,
          *refs):
        n = ROWS_PER_W
        story_v = refs[0:n]
        storyw_v = refs[n:2 * n]
        idxw_v = refs[2 * n:3 * n]
        gp_v = refs[3 * n:4 * n]
        gpw_v = refs[4 * n:5 * n]
        rows_v = refs[5 * n:6 * n]
        rowsw_v = refs[6 * n:7 * n]
        gpbuf = refs[7 * n:8 * n]
        ctx_v = refs[8 * n]
        sem_s = refs[8 * n + 1:8 * n + 1 + n]
        sem_w = refs[8 * n + 1 + n:8 * n + 1 + 2 * n]
        sem_g = refs[8 * n + 1 + 2 * n:8 * n + 1 + 3 * n]
        sem_r = refs[8 * n + 1 + 3 * n:8 * n + 1 + 4 * n]
        sem_o = refs[8 * n + 1 + 4 * n]

        cid = lax.axis_index("c")
        sid = lax.axis_index("s")
        w = sid * 2 + cid
        pltpu.sync_copy(ctx_hbm, ctx_v)
        zero16 = jnp.zeros((LANES,), jnp.float32)

        cps, cpw, cpg = [], [], []
        for r in range(ROWS_PER_W):
            b = w * ROWS_PER_W + r
            ctx16 = ctx_v[b]
            row0 = pl.multiple_of(b * M, LANES)
            cps.append(pltpu.async_copy(
                story_hbm.at[pl.ds(row0, KBW)], story_v[r], sem_s[r]))
            cpg.append(pltpu.async_copy(
                gp_hbm.at[pl.ds(row0, KBW)], gp_v[r], sem_g[r]))
            idxw_v[r][...] = b * M + ctx16 - 1
            cpw.append(pltpu.async_copy(
                story_hbm.at[idxw_v[r]], storyw_v[r], sem_w[r]))
            cpg.append(pltpu.async_copy(
                gp_hbm.at[idxw_v[r]], gpw_v[r], sem_g[r]))

        cpr, out_cps = [], []
        for r in range(ROWS_PER_W):
            cps[r].wait()
            cpr.append(pltpu.async_copy(c_hbm.at[story_v[r]], rows_v[r], sem_r[r]))
            cpw[r].wait()
            cpr.append(pltpu.async_copy(c_hbm.at[storyw_v[r]], rowsw_v[r], sem_r[r]))

        for r in range(ROWS_PER_W):
            b = w * ROWS_PER_W + r
            cpg[2 * r].wait()
            cpg[2 * r + 1].wait()
            for c in range(KBW // LANES):
                gpbuf[r][pl.ds(c * LANES, LANES)] = gp_v[r][pl.ds(c * LANES, LANES)]
            gpbuf[r][pl.ds(KBW, LANES)] = gpw_v[r][pl.ds(0, LANES)]
            for c in range(KBW // LANES + 1, 2 * KBW // LANES):
                gpbuf[r][pl.ds(c * LANES, LANES)] = zero16
            out_cps.append(pltpu.async_copy(gpbuf[r], gpp_hbm.at[b], sem_o))

        for r in range(ROWS_PER_W):
            b = w * ROWS_PER_W + r
            cpr[2 * r].wait()
            cpr[2 * r + 1].wait()
            out_cps.append(pltpu.async_copy(rows_v[r], rows_hbm.at[b], sem_o))
            out_cps.append(pltpu.async_copy(rowsw_v[r], roww_hbm.at[b], sem_o))

        for cp in out_cps:
            cp.wait()

    return k(c_know, story, gp, ctx_len)


def kernel(dh_outputs, dh_hidden, global_pointer, batch_size, story, domain,
           context_len, kb_len, conv_len, memory_mask, memory_story,
           W1, b1, W2, b2, C_know):
    i_vec = pl.pallas_call(
        _ivec_body,
        grid=(B // BBLK,),
        in_specs=[pl.BlockSpec((BBLK, S, D), lambda i: (i, 0, 0)),
                  pl.BlockSpec((1, BBLK, D), lambda i: (0, i, 0)),
                  pl.BlockSpec((2 * D, D), lambda i: (0, 0)),
                  pl.BlockSpec((1, D), lambda i: (0, 0)),
                  pl.BlockSpec((D, D), lambda i: (0, 0)),
                  pl.BlockSpec((1, D), lambda i: (0, 0))],
        out_specs=pl.BlockSpec((BBLK, D), lambda i: (i, 0)),
        out_shape=jax.ShapeDtypeStruct((B, D), jnp.float32),
    )(dh_outputs, dh_hidden, W1, b1.reshape(1, D), W2, b2.reshape(1, D))

    kb_b = jnp.broadcast_to(kb_len.astype(jnp.int32)[:, None], (B, LANES))
    ctx_b = jnp.broadcast_to(context_len.astype(jnp.int32)[:, None], (B, LANES))

    rows3, roww3, gpp = _sc_gather(C_know, story.reshape(B * M),
                                   global_pointer.reshape(B * M), ctx_b)

    toppi = pl.pallas_call(
        _final_body,
        out_shape=jax.ShapeDtypeStruct((B, TOPK), jnp.int32),
    )(rows3, roww3, gpp, kb_b, ctx_b, i_vec)
    return toppi, i_vec


# speedup vs baseline: 1.0044x; 1.0044x over previous
import functools

import jax
import jax.numpy as jnp
from jax import lax
from jax.experimental import pallas as pl
from jax.experimental.pallas import tpu as pltpu
from jax.experimental.pallas import tpu_sc as plsc

B, S, D, M, V = 64, 50, 128, 2048, 100000
TOPK = 12
NW = 32
ROWS_PER_W = B // NW
LANES = 16
KBW = 64


def _ivec_body(dh_ref, h_ref, w1_ref, b1_ref, w2_ref, b2_ref, out_ref):
    x = dh_ref[...]
    h = h_ref[0]
    hb = jnp.broadcast_to(h[:, None, :], (B, S, D))
    cat = jnp.concatenate([hb, x], axis=2).reshape(B * S, 2 * D)
    t = jnp.tanh(jnp.dot(cat, w1_ref[...],
                         preferred_element_type=jnp.float32) + b1_ref[...])
    q = (jnp.dot(t, w2_ref[...],
                 preferred_element_type=jnp.float32) + b2_ref[...])
    q = q.reshape(B, S, D)
    q = q - jnp.max(q, axis=1, keepdims=True)
    e = jnp.exp(q)
    q = e / jnp.sum(e, axis=1, keepdims=True)
    out_ref[...] = jnp.sum(q * x, axis=1)


def _pick_own_batch(R, n):
    R3 = R.reshape(B, n, B)
    bb = lax.broadcasted_iota(jnp.int32, (B, n, B), 0)
    jb = lax.broadcasted_iota(jnp.int32, (B, n, B), 2)
    return jnp.sum(jnp.where(jb == bb, R3, 0.0), axis=2)


def _final_body(rows_ref, roww_ref, gp_ref, kb_ref, ctx_ref, iv_ref, out_ref):
    iv = iv_ref[...]
    gpp = gp_ref[...]
    rowsS = rows_ref[...] * gpp[:, :KBW, None]
    rows2 = rowsS.reshape(B * KBW, D)
    R = lax.dot_general(rows2, iv, (((1,), (1,)), ((), ())))
    out64 = _pick_own_batch(R, KBW)
    rowwS = roww_ref[...] * gpp[:, KBW:KBW + 1, None]
    roww2 = rowwS.reshape(B * LANES, D)
    Rw = lax.dot_general(roww2, iv, (((1,), (1,)), ((), ())))
    outw = _pick_own_batch(Rw, LANES)[:, :1]

    kb = kb_ref[:, :1]
    ctx = ctx_ref[:, :1]
    pos64 = lax.broadcasted_iota(jnp.int32, (B, KBW), 1)
    bad64 = ((pos64 >= kb) & (pos64 < ctx - 1)) | (pos64 >= ctx)
    x64 = jnp.where(bad64, jnp.float32(-1e9), out64)
    sig64 = 1.0 / (1.0 + jnp.exp(-x64))
    win = 1.0 / (1.0 + jnp.exp(-outw))

    l = jnp.concatenate(
        [sig64, win, jnp.zeros((B, KBW - 1), jnp.float32)], axis=1)
    rawpos = lax.broadcasted_iota(jnp.int32, (B, 2 * KBW), 1)
    pos = jnp.where(rawpos == KBW, ctx - 1, rawpos)
    cols = []
    for _ in range(TOPK):
        v = jnp.max(l, axis=1, keepdims=True)
        idx = jnp.min(jnp.where(l == v, pos, M), axis=1, keepdims=True)
        cols.append(idx)
        l = jnp.where(pos == idx, -jnp.inf, l)
    out_ref[...] = jnp.concatenate(cols, axis=1)


def _sc_gather(c_know, story, gp, ctx_len):
    mesh = plsc.VectorSubcoreMesh(core_axis_name="c", subcore_axis_name="s")

    @functools.partial(
        pl.kernel, mesh=mesh,
        out_type=(jax.ShapeDtypeStruct((B, KBW, D), jnp.float32),
                  jax.ShapeDtypeStruct((B, LANES, D), jnp.float32),
                  jax.ShapeDtypeStruct((B, 2 * KBW), jnp.float32)),
        scratch_types=(
            [pltpu.VMEM((KBW,), jnp.int32)] * ROWS_PER_W
            + [pltpu.VMEM((LANES,), jnp.int32)] * ROWS_PER_W
            + [pltpu.VMEM((LANES,), jnp.int32)] * ROWS_PER_W
            + [pltpu.VMEM((KBW,), jnp.float32)] * ROWS_PER_W
            + [pltpu.VMEM((LANES,), jnp.float32)] * ROWS_PER_W
            + [pltpu.VMEM((KBW, D), jnp.float32)] * ROWS_PER_W
            + [pltpu.VMEM((LANES, D), jnp.float32)] * ROWS_PER_W
            + [pltpu.VMEM((2 * KBW,), jnp.float32)] * ROWS_PER_W
            + [pltpu.VMEM((B, LANES), jnp.int32)]
            + [pltpu.SemaphoreType.DMA] * (2 * ROWS_PER_W)
            + [pltpu.SemaphoreType.DMA] * ROWS_PER_W
            + [pltpu.SemaphoreType.DMA] * ROWS_PER_W
            + [pltpu.SemaphoreType.DMA]
        ),
    )
    def k(c_hbm, story_hbm, gp_hbm, ctx_hbm, rows_hbm, roww_hbm, gpp_hbm,
          *refs):
        n = ROWS_PER_W
        story_v = refs[0:n]
        storyw_v = refs[n:2 * n]
        idxw_v = refs[2 * n:3 * n]
        gp_v = refs[3 * n:4 * n]
        gpw_v = refs[4 * n:5 * n]
        rows_v = refs[5 * n:6 * n]
        rowsw_v = refs[6 * n:7 * n]
        gpbuf = refs[7 * n:8 * n]
        ctx_v = refs[8 * n]
        sem_s = refs[8 * n + 1:8 * n + 1 + n]
        sem_w = refs[8 * n + 1 + n:8 * n + 1 + 2 * n]
        sem_g = refs[8 * n + 1 + 2 * n:8 * n + 1 + 3 * n]
        sem_r = refs[8 * n + 1 + 3 * n:8 * n + 1 + 4 * n]
        sem_o = refs[8 * n + 1 + 4 * n]

        cid = lax.axis_index("c")
        sid = lax.axis_index("s")
        w = sid * 2 + cid
        pltpu.sync_copy(ctx_hbm, ctx_v)
        zero16 = jnp.zeros((LANES,), jnp.float32)

        cps, cpw, cpg = [], [], []
        for r in range(ROWS_PER_W):
            b = w * ROWS_PER_W + r
            ctx16 = ctx_v[b]
            row0 = pl.multiple_of(b * M, LANES)
            cps.append(pltpu.async_copy(
                story_hbm.at[pl.ds(row0, KBW)], story_v[r], sem_s[r]))
            cpg.append(pltpu.async_copy(
                gp_hbm.at[pl.ds(row0, KBW)], gp_v[r], sem_g[r]))
            idxw_v[r][...] = b * M + ctx16 - 1
            cpw.append(pltpu.async_copy(
                story_hbm.at[idxw_v[r]], storyw_v[r], sem_w[r]))
            cpg.append(pltpu.async_copy(
                gp_hbm.at[idxw_v[r]], gpw_v[r], sem_g[r]))

        cpr, out_cps = [], []
        for r in range(ROWS_PER_W):
            cps[r].wait()
            cpr.append(pltpu.async_copy(c_hbm.at[story_v[r]], rows_v[r], sem_r[r]))
            cpw[r].wait()
            cpr.append(pltpu.async_copy(c_hbm.at[storyw_v[r]], rowsw_v[r], sem_r[r]))

        for r in range(ROWS_PER_W):
            b = w * ROWS_PER_W + r
            cpg[2 * r].wait()
            cpg[2 * r + 1].wait()
            for c in range(KBW // LANES):
                gpbuf[r][pl.ds(c * LANES, LANES)] = gp_v[r][pl.ds(c * LANES, LANES)]
            gpbuf[r][pl.ds(KBW, LANES)] = gpw_v[r][pl.ds(0, LANES)]
            for c in range(KBW // LANES + 1, 2 * KBW // LANES):
                gpbuf[r][pl.ds(c * LANES, LANES)] = zero16
            out_cps.append(pltpu.async_copy(gpbuf[r], gpp_hbm.at[b], sem_o))

        for r in range(ROWS_PER_W):
            b = w * ROWS_PER_W + r
            cpr[2 * r].wait()
            cpr[2 * r + 1].wait()
            out_cps.append(pltpu.async_copy(rows_v[r], rows_hbm.at[b], sem_o))
            out_cps.append(pltpu.async_copy(rowsw_v[r], roww_hbm.at[b], sem_o))

        for cp in out_cps:
            cp.wait()

    return k(c_know, story, gp, ctx_len)


def kernel(dh_outputs, dh_hidden, global_pointer, batch_size, story, domain,
           context_len, kb_len, conv_len, memory_mask, memory_story,
           W1, b1, W2, b2, C_know):
    i_vec = pl.pallas_call(
        _ivec_body,
        out_shape=jax.ShapeDtypeStruct((B, D), jnp.float32),
    )(dh_outputs, dh_hidden, W1, b1.reshape(1, D), W2, b2.reshape(1, D))

    kb_b = jnp.broadcast_to(kb_len.astype(jnp.int32)[:, None], (B, LANES))
    ctx_b = jnp.broadcast_to(context_len.astype(jnp.int32)[:, None], (B, LANES))

    rows3, roww3, gpp = _sc_gather(C_know, story.reshape(B * M),
                                   global_pointer.reshape(B * M), ctx_b)

    toppi = pl.pallas_call(
        _final_body,
        out_shape=jax.ShapeDtypeStruct((B, TOPK), jnp.int32),
    )(rows3, roww3, gpp, kb_b, ctx_b, i_vec)
    return toppi, i_vec
